# baseline (speedup 1.0000x reference)
import functools
import math

import jax
import jax.numpy as jnp
from jax import lax
from jax.experimental import pallas as pl
from jax.experimental.pallas import tpu as pltpu
from jax.experimental.pallas import tpu_sc as plsc

_NT = 16
_NC = 2


_f32 = jnp.float32
_i32 = jnp.int32


def _mesh():
    return plsc.VectorSubcoreMesh(core_axis_name="c", subcore_axis_name="s")


@functools.lru_cache(maxsize=None)
def _make_outacc(E, NTAB, nseg, W, nsplit, dbuf=True):
    ept = E // _NT
    steps = ept // W
    assert ept % W == 0 and W % 8 == 0 and ept % 8 == 0 and W % 16 == 0
    half = nseg // nsplit
    assert nseg % nsplit == 0 and half % _NT == 0
    stripe = half // _NT
    zr = stripe if stripe <= 1280 else max(d for d in (1250, 1280, 640, 625, 400, 250, 125) if stripe % d == 0)
    nz = stripe // zr

    @functools.partial(
        pl.kernel,
        mesh=_mesh(),
        compiler_params=pltpu.CompilerParams(use_tc_tiling_on_sc=False, needs_layout_passes=False),
        out_type=jax.ShapeDtypeStruct((2, nseg, 16), _f32),
        scratch_types=[
            pltpu.VMEM((W,), _i32), pltpu.VMEM((W,), _i32),
            pltpu.VMEM((W,), _i32), pltpu.VMEM((W,), _i32),
            pltpu.VMEM((W,), _f32), pltpu.VMEM((W,), _f32),
            pltpu.VMEM((W, 16), _f32), pltpu.VMEM((W, 16), _f32),
            pltpu.VMEM((zr, 16), _f32),
            pltpu.VMEM_SHARED((half + 16, 16), _f32),
            pltpu.SemaphoreType.DMA, pltpu.SemaphoreType.DMA,
        ],
    )
    def k(vtab, srch, dsth, exh, zh, out, sidx0, sidx1, didx0, didx1,
          exv0, exv1, rows0, rows1, zbuf, acc, sem0, sem1):
        c = lax.axis_index("c")
        t = lax.axis_index("s")
        iota = lax.iota(_i32, 16)
        pltpu.sync_copy(zh, zbuf)
        base = c * NTAB
        bufs = ((sidx0, didx0, exv0, rows0, sem0),
                (sidx1, didx1, exv1, rows1, sem1))

        for ns in range(nsplit):
            lo = ns * half

            def zc(j, _):
                pltpu.sync_copy(zbuf, acc.at[pl.ds(t * stripe + j * zr, zr)])
                return 0

            lax.fori_loop(0, nz, zc, 0)
            plsc.subcore_barrier()

            def load(j, b):
                sidx, didx, exv, rows, sem = bufs[b]
                off = t * ept + j * W
                pltpu.sync_copy(srch.at[pl.ds(off, W)], sidx)
                pltpu.sync_copy(dsth.at[pl.ds(off, W)], didx)
                pltpu.sync_copy(exh.at[pl.ds(off, W)], exv)

                def adj(i, _):
                    sl = pl.ds(i * 16, 16)
                    sidx[sl] = sidx[sl] + base
                    if nsplit > 1:
                        d = didx[sl] - lo
                        ok = (d >= 0) & (d < half)
                        didx[sl] = jnp.where(ok, d, half + iota)
                    return 0

                lax.fori_loop(0, W // 16, adj, 0, unroll=8)
                pltpu.async_copy(vtab.at[sidx], rows, sem)

            def consume(b):
                sidx, didx, exv, rows, sem = bufs[b]
                pltpu.make_async_copy(vtab.at[sidx], rows, sem).wait()

                def scale(i, r):
                    es = plsc.load_gather(exv, [r])
                    rv = plsc.load_gather(rows, [r, iota])
                    plsc.store_scatter(rows, [r, iota], rv * es)
                    return r + 1

                lax.fori_loop(0, W, scale, jnp.zeros((16,), _i32), unroll=8)
                pltpu.sync_copy(rows, acc.at[didx], add=True)

            if dbuf:
                load(0, 0)

                def pair(j2, _):
                    j = j2 * 2
                    load(j + 1, 1)
                    consume(0)

                    @pl.when(j + 2 < steps)
                    def _():
                        load(j + 2, 0)

                    consume(1)
                    return 0

                lax.fori_loop(0, steps // 2, pair, 0)
                if steps % 2 == 1:
                    consume(0)
            else:
                def single(j, _):
                    load(j, 0)
                    consume(0)
                    return 0

                lax.fori_loop(0, steps, single, 0)
            plsc.subcore_barrier()
            pltpu.sync_copy(acc.at[pl.ds(t * stripe, stripe)],
                            out.at[c, pl.ds(lo + t * stripe, stripe)])
            if nsplit > 1 and ns + 1 < nsplit:
                plsc.subcore_barrier()

    def run(vtab, src, dst, ex):
        return k(vtab, src, dst, ex, jnp.zeros((zr, 16), _f32))

    return run


def _outacc(vtab_chunks, src, dst, ex, NTAB, nseg, W, nsplit, dbuf=True):
    nc = len(vtab_chunks)
    assert nc % 2 == 0
    outs = []
    for j in range(0, nc, 2):
        vt = jnp.concatenate([vtab_chunks[j], vtab_chunks[j + 1]], axis=0)
        outs.append(_make_outacc(src.shape[0], NTAB, nseg, W, nsplit, dbuf)(vt, src, dst, ex))
    return jnp.concatenate(outs, axis=0)


@functools.lru_cache(maxsize=None)
def _make_alpha(E, NTAB, W):
    ept = E // _NT
    steps = ept // W
    assert ept % W == 0 and W % 8 == 0 and ept % 8 == 0 and W % 16 == 0

    @functools.partial(
        pl.kernel,
        mesh=_mesh(),
        compiler_params=pltpu.CompilerParams(use_tc_tiling_on_sc=False, needs_layout_passes=False),
        out_type=jax.ShapeDtypeStruct((2, E), _f32),
        scratch_types=[
            pltpu.VMEM((W,), _i32), pltpu.VMEM((W,), _i32),
            pltpu.VMEM((W,), _i32), pltpu.VMEM((W,), _i32),
            pltpu.VMEM((W, 16), _f32), pltpu.VMEM((W, 16), _f32),
            pltpu.VMEM((W, 16), _f32), pltpu.VMEM((W, 16), _f32),
            pltpu.VMEM((W,), _f32),
            pltpu.SemaphoreType.DMA, pltpu.SemaphoreType.DMA,
        ],
    )
    def k(qtab, ktab, srch, dsth, out, qidx0, qidx1, kidx0, kidx1,
          qrows0, qrows1, krows0, krows1, pbuf, sem0, sem1):
        c = lax.axis_index("c")
        t = lax.axis_index("s")
        iota = lax.iota(_i32, 16)
        base = c * NTAB
        bufs = ((qidx0, kidx0, qrows0, krows0, sem0),
                (qidx1, kidx1, qrows1, krows1, sem1))

        def load(j, b):
            qidx, kidx, qrows, krows, sem = bufs[b]
            off = t * ept + j * W
            pltpu.sync_copy(dsth.at[pl.ds(off, W)], qidx)
            pltpu.sync_copy(srch.at[pl.ds(off, W)], kidx)

            def adj(i, _):
                qidx[pl.ds(i * 16, 16)] = qidx[pl.ds(i * 16, 16)] + base
                kidx[pl.ds(i * 16, 16)] = kidx[pl.ds(i * 16, 16)] + base
                return 0

            lax.fori_loop(0, W // 16, adj, 0, unroll=8)
            pltpu.async_copy(qtab.at[qidx], qrows, sem)
            pltpu.async_copy(ktab.at[kidx], krows, sem)

        def consume(j, b):
            qidx, kidx, qrows, krows, sem = bufs[b]
            off = t * ept + j * W
            pltpu.make_async_copy(qtab.at[qidx], qrows, sem).wait()
            pltpu.make_async_copy(ktab.at[kidx], krows, sem).wait()

            def dot(g, r):
                acc = jnp.zeros((16,), _f32)
                for f in range(16):
                    fidx = jnp.full((16,), f, _i32)
                    qv = plsc.load_gather(qrows, [r, fidx])
                    kv = plsc.load_gather(krows, [r, fidx])
                    acc = acc + qv * kv
                pbuf[pl.ds(g * 16, 16)] = acc
                return r + 16

            lax.fori_loop(0, W // 16, dot, iota)
            pltpu.sync_copy(pbuf, out.at[c, pl.ds(off, W)])

        load(0, 0)

        def pair(j2, _):
            j = j2 * 2
            load(j + 1, 1)
            consume(j, 0)

            @pl.when(j + 2 < steps)
            def _():
                load(j + 2, 0)

            consume(j + 1, 1)
            return 0

        lax.fori_loop(0, steps // 2, pair, 0)
        if steps % 2 == 1:
            consume(steps - 1, 0)

    return k


@functools.lru_cache(maxsize=None)
def _make_alpha2(E, NTAB, D, W):
    ept = E // (2 * _NT)
    steps = ept // W
    assert ept % W == 0 and W % 8 == 0 and ept % 8 == 0 and W % 16 == 0

    @functools.partial(
        pl.kernel,
        mesh=_mesh(),
        compiler_params=pltpu.CompilerParams(use_tc_tiling_on_sc=False, needs_layout_passes=False),
        out_type=jax.ShapeDtypeStruct((E,), _f32),
        scratch_types=[
            pltpu.VMEM((W,), _i32), pltpu.VMEM((W,), _i32),
            pltpu.VMEM((W,), _i32), pltpu.VMEM((W,), _i32),
            pltpu.VMEM((W, D), _f32), pltpu.VMEM((W, D), _f32),
            pltpu.VMEM((W, D), _f32), pltpu.VMEM((W, D), _f32),
            pltpu.VMEM((W,), _f32),
            pltpu.SemaphoreType.DMA, pltpu.SemaphoreType.DMA,
        ],
    )
    def k(qtab, ktab, srch, dsth, out, qidx0, qidx1, kidx0, kidx1,
          qrows0, qrows1, krows0, krows1, pbuf, sem0, sem1):
        c = lax.axis_index("c")
        t = lax.axis_index("s")
        w = t * 2 + c
        iota = lax.iota(_i32, 16)
        bufs = ((qidx0, kidx0, qrows0, krows0, sem0),
                (qidx1, kidx1, qrows1, krows1, sem1))

        def load(j, b):
            qidx, kidx, qrows, krows, sem = bufs[b]
            off = w * ept + j * W
            pltpu.sync_copy(dsth.at[pl.ds(off, W)], qidx)
            pltpu.sync_copy(srch.at[pl.ds(off, W)], kidx)
            pltpu.async_copy(qtab.at[qidx], qrows, sem)
            pltpu.async_copy(ktab.at[kidx], krows, sem)

        def consume(j, b):
            qidx, kidx, qrows, krows, sem = bufs[b]
            off = w * ept + j * W
            pltpu.make_async_copy(qtab.at[qidx], qrows, sem).wait()
            pltpu.make_async_copy(ktab.at[kidx], krows, sem).wait()

            def dot(g, r):
                acc = jnp.zeros((16,), _f32)
                for f in range(D):
                    fidx = jnp.full((16,), f, _i32)
                    qv = plsc.load_gather(qrows, [r, fidx])
                    kv = plsc.load_gather(krows, [r, fidx])
                    acc = acc + qv * kv
                pbuf[pl.ds(g * 16, 16)] = acc
                return r + 16

            lax.fori_loop(0, W // 16, dot, iota)
            pltpu.sync_copy(pbuf, out.at[pl.ds(off, W)])

        load(0, 0)

        def pair(j2, _):
            j = j2 * 2
            load(j + 1, 1)
            consume(j, 0)

            @pl.when(j + 2 < steps)
            def _():
                load(j + 2, 0)

            consume(j + 1, 1)
            return 0

        lax.fori_loop(0, steps // 2, pair, 0)
        if steps % 2 == 1:
            consume(steps - 1, 0)

    return k


def _alpha(q_chunks, k_chunks, src, dst, NTAB, W):
    nc = len(q_chunks)
    assert nc % 2 == 0 and nc == len(k_chunks)
    total = None
    for j in range(0, nc, 2):
        qt = jnp.concatenate([q_chunks[j], q_chunks[j + 1]], axis=0)
        kt = jnp.concatenate([k_chunks[j], k_chunks[j + 1]], axis=0)
        p = _make_alpha(src.shape[0], NTAB, W)(qt, kt, src, dst)
        s = p[0] + p[1]
        total = s if total is None else total + s
    return total


@functools.lru_cache(maxsize=None)
def _make_gather(B, D, W):
    ept = B // _NT
    steps = ept // W
    assert ept % W == 0 and W % 8 == 0 and ept % 8 == 0

    @functools.partial(
        pl.kernel,
        mesh=_mesh(),
        compiler_params=pltpu.CompilerParams(use_tc_tiling_on_sc=False, needs_layout_passes=False),
        out_type=jax.ShapeDtypeStruct((B, D), _f32),
        scratch_types=[
            pltpu.VMEM((W,), _i32),
            pltpu.VMEM((W, D), _f32),
            pltpu.SemaphoreType.DMA,
        ],
    )
    def k(tab, idxh, out, idxv, rows, sem):
        c = lax.axis_index("c")
        t = lax.axis_index("s")

        @pl.when(c == 0)
        def _():
            def step(j, _):
                off = t * ept + j * W
                pltpu.sync_copy(idxh.at[pl.ds(off, W)], idxv)
                pltpu.async_copy(tab.at[idxv], rows, sem).wait()
                pltpu.sync_copy(rows, out.at[pl.ds(off, W)])
                return 0

            lax.fori_loop(0, steps, step, 0)

    return k


def _mm_body(x_ref, w_ref, b_ref, o_ref, *, act):
    y = jnp.dot(x_ref[...], w_ref[...], preferred_element_type=_f32) + b_ref[...]
    if act == "relu":
        y = jnp.maximum(y, 0.0)
    o_ref[...] = y


@functools.lru_cache(maxsize=None)
def _make_mm(N, F, K, act, BR):
    assert N % BR == 0
    return pl.pallas_call(
        functools.partial(_mm_body, act=act),
        grid=(N // BR,),
        in_specs=[pl.BlockSpec((BR, F), lambda i: (i, 0)),
                  pl.BlockSpec((F, K), lambda i: (0, 0)),
                  pl.BlockSpec((1, K), lambda i: (0, 0))],
        out_specs=pl.BlockSpec((BR, K), lambda i: (i, 0)),
        out_shape=jax.ShapeDtypeStruct((N, K), _f32),
    )


def _mm(x, w, b, act=None, BR=2000):
    return _make_mm(x.shape[0], x.shape[1], w.shape[1], act, BR)(x, w, b.reshape(1, -1))


def _exp_body(p_ref, o_ref, *, scale, e_true, B):
    a = jnp.sum(p_ref[...], axis=0, keepdims=True) * scale
    ex = jnp.exp(jnp.minimum(a, 60.0))
    if e_true is not None:
        i = pl.program_id(0)
        idx = i * B + jax.lax.broadcasted_iota(_i32, (1, B), 1)
        ex = jnp.where(idx < e_true, ex, 0.0)
    o_ref[...] = ex


@functools.lru_cache(maxsize=None)
def _make_exp(nc, E, scale, e_true, B):
    return pl.pallas_call(
        functools.partial(_exp_body, scale=scale, e_true=e_true, B=B),
        grid=(E // B,),
        in_specs=[pl.BlockSpec((nc, B), lambda i: (0, i))],
        out_specs=pl.BlockSpec((1, B), lambda i: (0, i)),
        out_shape=jax.ShapeDtypeStruct((1, E), _f32),
    )


def _comb_body(acc_ref, xr_ref, wo_ref, wr_ref, o_ref, *, cfeat):
    acc = acc_ref[...]
    nc = acc.shape[0]
    feats = jnp.concatenate([acc[j, :, :15] for j in range(nc)], axis=1)[:, :cfeat]
    den = acc[0, :, 15:16]
    out = feats / (den + 1e-16)
    xr = xr_ref[...]
    beta = jax.nn.sigmoid(out @ wo_ref[...] + xr @ wr_ref[...])
    o_ref[...] = jnp.maximum(beta * xr + (1.0 - beta) * out, 0.0)


@functools.lru_cache(maxsize=None)
def _make_comb(nc, N, cfeat, BR):
    return pl.pallas_call(
        functools.partial(_comb_body, cfeat=cfeat),
        grid=(N // BR,),
        in_specs=[pl.BlockSpec((nc, BR, 16), lambda i: (0, i, 0)),
                  pl.BlockSpec((BR, cfeat), lambda i: (i, 0)),
                  pl.BlockSpec((cfeat, 1), lambda i: (0, 0)),
                  pl.BlockSpec((cfeat, 1), lambda i: (0, 0))],
        out_specs=pl.BlockSpec((BR, cfeat), lambda i: (i, 0)),
        out_shape=jax.ShapeDtypeStruct((N, cfeat), _f32),
    )


def _scale_body(b_ref, v_ref, o_ref):
    o_ref[...] = b_ref[...] * v_ref[...]


@functools.lru_cache(maxsize=None)
def _make_scalerows(N, C):
    return pl.pallas_call(
        _scale_body,
        in_specs=[pl.BlockSpec((N, C), lambda: (0, 0)),
                  pl.BlockSpec((N, 1), lambda: (0, 0))],
        out_specs=pl.BlockSpec((N, C), lambda: (0, 0)),
        out_shape=jax.ShapeDtypeStruct((N, C), _f32),
    )


def _head_body(acc_ref, inc_ref, ta_ref, w1_ref, b1_ref, w2_ref, b2_ref, o_ref, *, cfeat):
    acc = acc_ref[...]
    nc = acc.shape[0]
    feats = jnp.concatenate([acc[j, :, :15] for j in range(nc)], axis=1)[:, :cfeat]
    cnt = acc[0, :, 15:16]
    xm = feats / jnp.maximum(cnt, 1.0)
    h = jnp.concatenate([xm, inc_ref[...], ta_ref[...]], axis=1)
    h = jnp.maximum(h @ w1_ref[...] + b1_ref[...], 0.0)
    o_ref[...] = jnp.tanh(h @ w2_ref[...] + b2_ref[...])


@functools.lru_cache(maxsize=None)
def _make_head(nc, Gn, cfeat, F1, F2):
    return pl.pallas_call(
        functools.partial(_head_body, cfeat=cfeat),
        in_specs=[pl.BlockSpec((nc, Gn, 16), lambda: (0, 0, 0)),
                  pl.BlockSpec((Gn, 4), lambda: (0, 0)),
                  pl.BlockSpec((Gn, 1), lambda: (0, 0)),
                  pl.BlockSpec((F1, F2), lambda: (0, 0)),
                  pl.BlockSpec((1, F2), lambda: (0, 0)),
                  pl.BlockSpec((F2, 1), lambda: (0, 0)),
                  pl.BlockSpec((1, 1), lambda: (0, 0))],
        out_specs=pl.BlockSpec((Gn, 1), lambda: (0, 0)),
        out_shape=jax.ShapeDtypeStruct((Gn, 1), _f32),
    )


def _pack_qk(q):
    n, cfeat = q.shape
    nc = -(-cfeat // 16)
    nc = nc + (nc % 2)
    qp = jnp.pad(q, ((0, 0), (0, nc * 16 - cfeat)))
    return jnp.split(qp, nc, axis=1)


def _pack_v(v, with_ones=True):
    n, cfeat = v.shape
    nc = -(-cfeat // 15)
    chunks = []
    for j in range(nc):
        blk = v[:, 15 * j:15 * (j + 1)]
        blk = jnp.pad(blk, ((0, 0), (0, 15 - blk.shape[1])))
        col = jnp.ones((n, 1), _f32) if (with_ones and j == 0) else jnp.zeros((n, 1), _f32)
        chunks.append(jnp.concatenate([blk, col], axis=1))
    if len(chunks) % 2:
        chunks.append(jnp.zeros((n, 16), _f32))
    return chunks


def _unpack_acc(acc, cfeat):
    nc = acc.shape[0]
    feats = jnp.concatenate([acc[j, :, :15] for j in range(nc)], axis=1)[:, :cfeat]
    return feats, acc[0, :, 15]


def _pad_edges(src, dst, ex, nseg, mult):
    e = src.shape[0]
    ep = -(-e // mult) * mult
    if ep == e:
        return src, dst, ex, e
    p = ep - e
    pad_dst = (jnp.arange(p, dtype=_i32) % nseg)
    src = jnp.concatenate([src.astype(_i32), jnp.zeros((p,), _i32)])
    dst = jnp.concatenate([dst.astype(_i32), pad_dst])
    ex = jnp.concatenate([ex, jnp.zeros((p,), _f32)])
    return src, dst, ex, ep


def _tconv_sc(x, src, dst, p, W_alpha, W_out, e_true=None):
    n = x.shape[0]
    cfeat = p['Wq'].shape[1]
    wcat = jnp.concatenate([p['Wq'], p['Wk'], p['Wv'], p['Ws']], axis=1)
    bcat = jnp.concatenate([p['bq'], p['bk'], p['bv'], p['bs']])
    qkvs = _mm(x, wcat, bcat, act=None)
    q = qkvs[:, :cfeat]
    k = qkvs[:, cfeat:2 * cfeat]
    v = qkvs[:, 2 * cfeat:3 * cfeat]
    x_r = qkvs[:, 3 * cfeat:]

    E = src.shape[0]
    D = 32 if cfeat <= 32 else 64
    qt = jnp.pad(q, ((0, 0), (0, D - cfeat)))
    kt = jnp.pad(k, ((0, 0), (0, D - cfeat)))
    dots = _make_alpha2(E, n, D, W_alpha)(qt, kt, src, dst)
    scale = 1.0 / math.sqrt(cfeat)
    BEXP = next(b for b in (12800, 2048, 1024, 512) if E % b == 0)
    ex = _make_exp(1, E, scale, e_true, BEXP)(dots.reshape(1, E)).reshape(E)

    acc = _outacc(_pack_v(v, with_ones=True), src, dst, ex, n, n, W_out,
                  nsplit=1 if n > 50000 else 2, dbuf=n <= 50000)
    wo = (p['Wb'][:cfeat] + p['Wb'][2 * cfeat:]).reshape(cfeat, 1)
    wr = (p['Wb'][cfeat:2 * cfeat] - p['Wb'][2 * cfeat:]).reshape(cfeat, 1)
    return _make_comb(acc.shape[0], n, cfeat, 2000)(acc, x_r, wo, wr)


def _bonus_block_sc(xtab, n, bonus_nodes_p, bsrc, bdst, eb_true, bonus_batch, bm_row,
                    bm_col, bonus_values_normed, pb, BN, NB):
    BNp = bonus_nodes_p.shape[0]
    xb = _make_gather(BNp, 32, 1280)(xtab, bonus_nodes_p)[:BN, :20]
    b = _tconv_sc(xb, bsrc, bdst, pb, 640, 1280, e_true=eb_true)

    psrc, pdst, pex, BPE = _pad_edges(jnp.arange(BN, dtype=_i32), bonus_batch,
                                      jnp.ones((BN,), _f32), NB, 16 * 1280)
    pool = _outacc(_pack_v(b, with_ones=False), psrc, pdst, pex, BN, NB, 1280, nsplit=1)
    bpool, _ = _unpack_acc(pool, 20)

    vt = _make_scalerows(NB, 20)(bpool, bonus_values_normed.reshape(NB, 1))
    ssrc, sdst, sex, SPE = _pad_edges(bm_col, bm_row, jnp.ones((bm_col.shape[0],), _f32),
                                      n, 16 * 1280)
    sacc = _outacc(_pack_v(vt, with_ones=False), ssrc, sdst, sex, NB, n, 1280, nsplit=25)
    bres, _ = _unpack_acc(sacc, 20)
    return bres


def _pad32(x):
    return jnp.pad(x, ((0, 0), (0, 32 - x.shape[1])))


def kernel(graph_features, graph_edges, bonus_nodes, bonus_edges, bonus_batch, bonus_mapping,
           bonus_values_normed, batch, income, total_armies, params):
    N = graph_features.shape[0]
    BN = bonus_nodes.shape[0]
    NB = bonus_values_normed.shape[0]
    Gn = income.shape[0]

    gsrc = graph_edges[0].astype(_i32)
    gdst = graph_edges[1].astype(_i32)
    eb_true = bonus_edges.shape[1]
    ebp = -(-eb_true // 20480) * 20480 - eb_true
    bsrc = jnp.concatenate([bonus_edges[0].astype(_i32), jnp.zeros((ebp,), _i32)])
    bdst = jnp.concatenate([bonus_edges[1].astype(_i32), jnp.zeros((ebp,), _i32)])
    bm_row = bonus_mapping[0].astype(_i32)
    bm_col = bonus_mapping[1].astype(_i32)
    bnp = jnp.concatenate([bonus_nodes.astype(_i32),
                           jnp.zeros((20480 - BN,), _i32)])

    x = _mm(graph_features, params['init_W'], params['init_b'], act="relu")

    for li, (pb, pg) in enumerate([(params['b1'], params['g1']),
                                   (params['b2'], params['g2']),
                                   (params['b3'], params['g3'])]):
        bres = _bonus_block_sc(_pad32(x), N, bnp, bsrc, bdst, eb_true, bonus_batch,
                               bm_row, bm_col, bonus_values_normed, pb, BN, NB)
        xc = jnp.concatenate([x, bres], axis=1)
        x = _tconv_sc(xc, gsrc, gdst, pg, 400, 400)

    psrc, pdst, pex, PE = _pad_edges(jnp.arange(N, dtype=_i32), batch,
                                     jnp.ones((N,), _f32), Gn, 16 * 1600)
    pool = _outacc(_pack_v(x, with_ones=True), psrc, pdst, pex, N, Gn, 1600, nsplit=1)
    out = _make_head(pool.shape[0], Gn, x.shape[1], 65, 60)(
        pool, income, total_armies, params['f1_W'], params['f1_b'].reshape(1, -1),
        params['f2_W'], params['f2_b'].reshape(1, 1)).reshape(-1)
    pi = jax.nn.log_softmax(jnp.zeros((Gn, 50), dtype=_f32), axis=1)
    return (out, pi)


# speedup vs baseline: 1.1136x; 1.1136x over previous
import functools
import math

import jax
import jax.numpy as jnp
from jax import lax
from jax.experimental import pallas as pl
from jax.experimental.pallas import tpu as pltpu
from jax.experimental.pallas import tpu_sc as plsc

_NT = 16
_NC = 2


_f32 = jnp.float32
_i32 = jnp.int32


def _mesh():
    return plsc.VectorSubcoreMesh(core_axis_name="c", subcore_axis_name="s")


@functools.lru_cache(maxsize=None)
def _make_outacc(E, NTAB, nseg, W, nsplit, dbuf=True):
    ept = E // _NT
    steps = ept // W
    assert ept % W == 0 and W % 8 == 0 and ept % 8 == 0 and W % 16 == 0
    half = nseg // nsplit
    assert nseg % nsplit == 0 and half % _NT == 0
    stripe = half // _NT
    zr = stripe if stripe <= 1280 else max(d for d in (1250, 1280, 640, 625, 400, 250, 125) if stripe % d == 0)
    nz = stripe // zr

    @functools.partial(
        pl.kernel,
        mesh=_mesh(),
        compiler_params=pltpu.CompilerParams(use_tc_tiling_on_sc=False, needs_layout_passes=False),
        out_type=jax.ShapeDtypeStruct((2, nseg, 16), _f32),
        scratch_types=[
            pltpu.VMEM((W,), _i32), pltpu.VMEM((W,), _i32),
            pltpu.VMEM((W,), _i32), pltpu.VMEM((W,), _i32),
            pltpu.VMEM((W,), _f32), pltpu.VMEM((W,), _f32),
            pltpu.VMEM((W, 16), _f32), pltpu.VMEM((W, 16), _f32),
            pltpu.VMEM((zr, 16), _f32),
            pltpu.VMEM_SHARED((half + 16, 16), _f32),
            pltpu.SemaphoreType.DMA, pltpu.SemaphoreType.DMA,
        ],
    )
    def k(vtab, srch, dsth, exh, zh, out, sidx0, sidx1, didx0, didx1,
          exv0, exv1, rows0, rows1, zbuf, acc, sem0, sem1):
        c = lax.axis_index("c")
        t = lax.axis_index("s")
        iota = lax.iota(_i32, 16)
        pltpu.sync_copy(zh, zbuf)
        base = c * NTAB
        bufs = ((sidx0, didx0, exv0, rows0, sem0),
                (sidx1, didx1, exv1, rows1, sem1))

        for ns in range(nsplit):
            lo = ns * half

            def zc(j, _):
                pltpu.sync_copy(zbuf, acc.at[pl.ds(t * stripe + j * zr, zr)])
                return 0

            lax.fori_loop(0, nz, zc, 0)
            plsc.subcore_barrier()

            def load(j, b):
                sidx, didx, exv, rows, sem = bufs[b]
                off = t * ept + j * W
                pltpu.sync_copy(srch.at[pl.ds(off, W)], sidx)
                pltpu.sync_copy(dsth.at[pl.ds(off, W)], didx)
                pltpu.sync_copy(exh.at[pl.ds(off, W)], exv)

                def adj(i, _):
                    sl = pl.ds(i * 16, 16)
                    sidx[sl] = sidx[sl] + base
                    if nsplit > 1:
                        d = didx[sl] - lo
                        ok = (d >= 0) & (d < half)
                        didx[sl] = jnp.where(ok, d, half + iota)
                    return 0

                lax.fori_loop(0, W // 16, adj, 0, unroll=8)
                pltpu.async_copy(vtab.at[sidx], rows, sem)

            def consume(b):
                sidx, didx, exv, rows, sem = bufs[b]
                pltpu.make_async_copy(vtab.at[sidx], rows, sem).wait()

                def scale(i, r):
                    es = plsc.load_gather(exv, [r])
                    rv = plsc.load_gather(rows, [r, iota])
                    plsc.store_scatter(rows, [r, iota], rv * es)
                    return r + 1

                lax.fori_loop(0, W, scale, jnp.zeros((16,), _i32), unroll=8)
                pltpu.sync_copy(rows, acc.at[didx], add=True)

            if dbuf:
                load(0, 0)

                def pair(j2, _):
                    j = j2 * 2
                    load(j + 1, 1)
                    consume(0)

                    @pl.when(j + 2 < steps)
                    def _():
                        load(j + 2, 0)

                    consume(1)
                    return 0

                lax.fori_loop(0, steps // 2, pair, 0)
                if steps % 2 == 1:
                    consume(0)
            else:
                def single(j, _):
                    load(j, 0)
                    consume(0)
                    return 0

                lax.fori_loop(0, steps, single, 0)
            plsc.subcore_barrier()
            pltpu.sync_copy(acc.at[pl.ds(t * stripe, stripe)],
                            out.at[c, pl.ds(lo + t * stripe, stripe)])
            if nsplit > 1 and ns + 1 < nsplit:
                plsc.subcore_barrier()

    def run(vtab, src, dst, ex):
        return k(vtab, src, dst, ex, jnp.zeros((zr, 16), _f32))

    return run


def _outacc(vtab_chunks, src, dst, ex, NTAB, nseg, W, nsplit, dbuf=True):
    nc = len(vtab_chunks)
    assert nc % 2 == 0
    outs = []
    for j in range(0, nc, 2):
        vt = jnp.concatenate([vtab_chunks[j], vtab_chunks[j + 1]], axis=0)
        outs.append(_make_outacc(src.shape[0], NTAB, nseg, W, nsplit, dbuf)(vt, src, dst, ex))
    return jnp.concatenate(outs, axis=0)


@functools.lru_cache(maxsize=None)
def _make_alpha(E, NTAB, W):
    ept = E // _NT
    steps = ept // W
    assert ept % W == 0 and W % 8 == 0 and ept % 8 == 0 and W % 16 == 0

    @functools.partial(
        pl.kernel,
        mesh=_mesh(),
        compiler_params=pltpu.CompilerParams(use_tc_tiling_on_sc=False, needs_layout_passes=False),
        out_type=jax.ShapeDtypeStruct((2, E), _f32),
        scratch_types=[
            pltpu.VMEM((W,), _i32), pltpu.VMEM((W,), _i32),
            pltpu.VMEM((W,), _i32), pltpu.VMEM((W,), _i32),
            pltpu.VMEM((W, 16), _f32), pltpu.VMEM((W, 16), _f32),
            pltpu.VMEM((W, 16), _f32), pltpu.VMEM((W, 16), _f32),
            pltpu.VMEM((W,), _f32),
            pltpu.SemaphoreType.DMA, pltpu.SemaphoreType.DMA,
        ],
    )
    def k(qtab, ktab, srch, dsth, out, qidx0, qidx1, kidx0, kidx1,
          qrows0, qrows1, krows0, krows1, pbuf, sem0, sem1):
        c = lax.axis_index("c")
        t = lax.axis_index("s")
        iota = lax.iota(_i32, 16)
        base = c * NTAB
        bufs = ((qidx0, kidx0, qrows0, krows0, sem0),
                (qidx1, kidx1, qrows1, krows1, sem1))

        def load(j, b):
            qidx, kidx, qrows, krows, sem = bufs[b]
            off = t * ept + j * W
            pltpu.sync_copy(dsth.at[pl.ds(off, W)], qidx)
            pltpu.sync_copy(srch.at[pl.ds(off, W)], kidx)

            def adj(i, _):
                qidx[pl.ds(i * 16, 16)] = qidx[pl.ds(i * 16, 16)] + base
                kidx[pl.ds(i * 16, 16)] = kidx[pl.ds(i * 16, 16)] + base
                return 0

            lax.fori_loop(0, W // 16, adj, 0, unroll=8)
            pltpu.async_copy(qtab.at[qidx], qrows, sem)
            pltpu.async_copy(ktab.at[kidx], krows, sem)

        def consume(j, b):
            qidx, kidx, qrows, krows, sem = bufs[b]
            off = t * ept + j * W
            pltpu.make_async_copy(qtab.at[qidx], qrows, sem).wait()
            pltpu.make_async_copy(ktab.at[kidx], krows, sem).wait()

            def dot(g, r):
                acc = jnp.zeros((16,), _f32)
                for f in range(16):
                    fidx = jnp.full((16,), f, _i32)
                    qv = plsc.load_gather(qrows, [r, fidx])
                    kv = plsc.load_gather(krows, [r, fidx])
                    acc = acc + qv * kv
                pbuf[pl.ds(g * 16, 16)] = acc
                return r + 16

            lax.fori_loop(0, W // 16, dot, iota)
            pltpu.sync_copy(pbuf, out.at[c, pl.ds(off, W)])

        load(0, 0)

        def pair(j2, _):
            j = j2 * 2
            load(j + 1, 1)
            consume(j, 0)

            @pl.when(j + 2 < steps)
            def _():
                load(j + 2, 0)

            consume(j + 1, 1)
            return 0

        lax.fori_loop(0, steps // 2, pair, 0)
        if steps % 2 == 1:
            consume(steps - 1, 0)

    return k


def _alpha(q_chunks, k_chunks, src, dst, NTAB, W):
    nc = len(q_chunks)
    assert nc % 2 == 0 and nc == len(k_chunks)
    total = None
    for j in range(0, nc, 2):
        qt = jnp.concatenate([q_chunks[j], q_chunks[j + 1]], axis=0)
        kt = jnp.concatenate([k_chunks[j], k_chunks[j + 1]], axis=0)
        p = _make_alpha(src.shape[0], NTAB, W)(qt, kt, src, dst)
        s = p[0] + p[1]
        total = s if total is None else total + s
    return total


@functools.lru_cache(maxsize=None)
def _make_gather(B, D, W):
    ept = B // _NT
    steps = ept // W
    assert ept % W == 0 and W % 8 == 0 and ept % 8 == 0

    @functools.partial(
        pl.kernel,
        mesh=_mesh(),
        compiler_params=pltpu.CompilerParams(use_tc_tiling_on_sc=False, needs_layout_passes=False),
        out_type=jax.ShapeDtypeStruct((B, D), _f32),
        scratch_types=[
            pltpu.VMEM((W,), _i32),
            pltpu.VMEM((W, D), _f32),
            pltpu.SemaphoreType.DMA,
        ],
    )
    def k(tab, idxh, out, idxv, rows, sem):
        c = lax.axis_index("c")
        t = lax.axis_index("s")

        @pl.when(c == 0)
        def _():
            def step(j, _):
                off = t * ept + j * W
                pltpu.sync_copy(idxh.at[pl.ds(off, W)], idxv)
                pltpu.async_copy(tab.at[idxv], rows, sem).wait()
                pltpu.sync_copy(rows, out.at[pl.ds(off, W)])
                return 0

            lax.fori_loop(0, steps, step, 0)

    return k


def _mm_body(x_ref, w_ref, b_ref, o_ref, *, act):
    y = jnp.dot(x_ref[...], w_ref[...], preferred_element_type=_f32) + b_ref[...]
    if act == "relu":
        y = jnp.maximum(y, 0.0)
    o_ref[...] = y


@functools.lru_cache(maxsize=None)
def _make_mm(N, F, K, act, BR):
    assert N % BR == 0
    return pl.pallas_call(
        functools.partial(_mm_body, act=act),
        grid=(N // BR,),
        in_specs=[pl.BlockSpec((BR, F), lambda i: (i, 0)),
                  pl.BlockSpec((F, K), lambda i: (0, 0)),
                  pl.BlockSpec((1, K), lambda i: (0, 0))],
        out_specs=pl.BlockSpec((BR, K), lambda i: (i, 0)),
        out_shape=jax.ShapeDtypeStruct((N, K), _f32),
    )


def _mm(x, w, b, act=None, BR=2000):
    return _make_mm(x.shape[0], x.shape[1], w.shape[1], act, BR)(x, w, b.reshape(1, -1))


def _exp_body(p_ref, o_ref, *, scale, e_true, B):
    a = jnp.sum(p_ref[...], axis=0, keepdims=True) * scale
    ex = jnp.exp(jnp.minimum(a, 60.0))
    if e_true is not None:
        i = pl.program_id(0)
        idx = i * B + jax.lax.broadcasted_iota(_i32, (1, B), 1)
        ex = jnp.where(idx < e_true, ex, 0.0)
    o_ref[...] = ex


@functools.lru_cache(maxsize=None)
def _make_exp(nc, E, scale, e_true, B):
    return pl.pallas_call(
        functools.partial(_exp_body, scale=scale, e_true=e_true, B=B),
        grid=(E // B,),
        in_specs=[pl.BlockSpec((nc, B), lambda i: (0, i))],
        out_specs=pl.BlockSpec((1, B), lambda i: (0, i)),
        out_shape=jax.ShapeDtypeStruct((1, E), _f32),
    )


def _comb_body(acc_ref, xr_ref, wo_ref, wr_ref, o_ref, *, cfeat):
    acc = acc_ref[...]
    nc = acc.shape[0]
    feats = jnp.concatenate([acc[j, :, :15] for j in range(nc)], axis=1)[:, :cfeat]
    den = acc[0, :, 15:16]
    out = feats / (den + 1e-16)
    xr = xr_ref[...]
    beta = jax.nn.sigmoid(out @ wo_ref[...] + xr @ wr_ref[...])
    o_ref[...] = jnp.maximum(beta * xr + (1.0 - beta) * out, 0.0)


@functools.lru_cache(maxsize=None)
def _make_comb(nc, N, cfeat, BR):
    return pl.pallas_call(
        functools.partial(_comb_body, cfeat=cfeat),
        grid=(N // BR,),
        in_specs=[pl.BlockSpec((nc, BR, 16), lambda i: (0, i, 0)),
                  pl.BlockSpec((BR, cfeat), lambda i: (i, 0)),
                  pl.BlockSpec((cfeat, 1), lambda i: (0, 0)),
                  pl.BlockSpec((cfeat, 1), lambda i: (0, 0))],
        out_specs=pl.BlockSpec((BR, cfeat), lambda i: (i, 0)),
        out_shape=jax.ShapeDtypeStruct((N, cfeat), _f32),
    )


def _scale_body(b_ref, v_ref, o_ref):
    o_ref[...] = b_ref[...] * v_ref[...]


@functools.lru_cache(maxsize=None)
def _make_scalerows(N, C):
    return pl.pallas_call(
        _scale_body,
        in_specs=[pl.BlockSpec((N, C), lambda: (0, 0)),
                  pl.BlockSpec((N, 1), lambda: (0, 0))],
        out_specs=pl.BlockSpec((N, C), lambda: (0, 0)),
        out_shape=jax.ShapeDtypeStruct((N, C), _f32),
    )


def _head_body(acc_ref, inc_ref, ta_ref, w1_ref, b1_ref, w2_ref, b2_ref, o_ref, *, cfeat):
    acc = acc_ref[...]
    nc = acc.shape[0]
    feats = jnp.concatenate([acc[j, :, :15] for j in range(nc)], axis=1)[:, :cfeat]
    cnt = acc[0, :, 15:16]
    xm = feats / jnp.maximum(cnt, 1.0)
    h = jnp.concatenate([xm, inc_ref[...], ta_ref[...]], axis=1)
    h = jnp.maximum(h @ w1_ref[...] + b1_ref[...], 0.0)
    o_ref[...] = jnp.tanh(h @ w2_ref[...] + b2_ref[...])


@functools.lru_cache(maxsize=None)
def _make_head(nc, Gn, cfeat, F1, F2):
    return pl.pallas_call(
        functools.partial(_head_body, cfeat=cfeat),
        in_specs=[pl.BlockSpec((nc, Gn, 16), lambda: (0, 0, 0)),
                  pl.BlockSpec((Gn, 4), lambda: (0, 0)),
                  pl.BlockSpec((Gn, 1), lambda: (0, 0)),
                  pl.BlockSpec((F1, F2), lambda: (0, 0)),
                  pl.BlockSpec((1, F2), lambda: (0, 0)),
                  pl.BlockSpec((F2, 1), lambda: (0, 0)),
                  pl.BlockSpec((1, 1), lambda: (0, 0))],
        out_specs=pl.BlockSpec((Gn, 1), lambda: (0, 0)),
        out_shape=jax.ShapeDtypeStruct((Gn, 1), _f32),
    )


def _pack_qk(q):
    n, cfeat = q.shape
    nc = -(-cfeat // 16)
    nc = nc + (nc % 2)
    qp = jnp.pad(q, ((0, 0), (0, nc * 16 - cfeat)))
    return jnp.split(qp, nc, axis=1)


def _pack_v(v, with_ones=True):
    n, cfeat = v.shape
    nc = -(-cfeat // 15)
    chunks = []
    for j in range(nc):
        blk = v[:, 15 * j:15 * (j + 1)]
        blk = jnp.pad(blk, ((0, 0), (0, 15 - blk.shape[1])))
        col = jnp.ones((n, 1), _f32) if (with_ones and j == 0) else jnp.zeros((n, 1), _f32)
        chunks.append(jnp.concatenate([blk, col], axis=1))
    if len(chunks) % 2:
        chunks.append(jnp.zeros((n, 16), _f32))
    return chunks


def _unpack_acc(acc, cfeat):
    nc = acc.shape[0]
    feats = jnp.concatenate([acc[j, :, :15] for j in range(nc)], axis=1)[:, :cfeat]
    return feats, acc[0, :, 15]


def _pad_edges(src, dst, ex, nseg, mult):
    e = src.shape[0]
    ep = -(-e // mult) * mult
    if ep == e:
        return src, dst, ex, e
    p = ep - e
    pad_dst = (jnp.arange(p, dtype=_i32) % nseg)
    src = jnp.concatenate([src.astype(_i32), jnp.zeros((p,), _i32)])
    dst = jnp.concatenate([dst.astype(_i32), pad_dst])
    ex = jnp.concatenate([ex, jnp.zeros((p,), _f32)])
    return src, dst, ex, ep


def _tconv_sc(x, src, dst, p, W_alpha, W_out, e_true=None):
    n = x.shape[0]
    cfeat = p['Wq'].shape[1]
    wcat = jnp.concatenate([p['Wq'], p['Wk'], p['Wv'], p['Ws']], axis=1)
    bcat = jnp.concatenate([p['bq'], p['bk'], p['bv'], p['bs']])
    qkvs = _mm(x, wcat, bcat, act=None)
    q = qkvs[:, :cfeat]
    k = qkvs[:, cfeat:2 * cfeat]
    v = qkvs[:, 2 * cfeat:3 * cfeat]
    x_r = qkvs[:, 3 * cfeat:]

    E = src.shape[0]
    qcs = _pack_qk(q)
    kcs = _pack_qk(k)
    nc = len(qcs)
    parts = []
    for j in range(0, nc, 2):
        qt = jnp.concatenate([qcs[j], qcs[j + 1]], axis=0)
        kt = jnp.concatenate([kcs[j], kcs[j + 1]], axis=0)
        parts.append(_make_alpha(E, n, W_alpha)(qt, kt, src, dst))
    partial = jnp.concatenate(parts, axis=0)
    scale = 1.0 / math.sqrt(cfeat)
    BEXP = next(b for b in (12800, 2048, 1024, 512) if E % b == 0)
    ex = _make_exp(nc, E, scale, e_true, BEXP)(partial).reshape(E)

    acc = _outacc(_pack_v(v, with_ones=True), src, dst, ex, n, n, W_out,
                  nsplit=1 if n > 50000 else 2, dbuf=n <= 50000)
    wo = (p['Wb'][:cfeat] + p['Wb'][2 * cfeat:]).reshape(cfeat, 1)
    wr = (p['Wb'][cfeat:2 * cfeat] - p['Wb'][2 * cfeat:]).reshape(cfeat, 1)
    return _make_comb(acc.shape[0], n, cfeat, 2000)(acc, x_r, wo, wr)


def _bonus_block_sc(xtab, n, bonus_nodes_p, bsrc, bdst, eb_true, bonus_batch, bm_row,
                    bm_col, bonus_values_normed, pb, BN, NB):
    BNp = bonus_nodes_p.shape[0]
    xb = _make_gather(BNp, 32, 1280)(xtab, bonus_nodes_p)[:BN, :20]
    b = _tconv_sc(xb, bsrc, bdst, pb, 1280, 1280, e_true=eb_true)

    psrc, pdst, pex, BPE = _pad_edges(jnp.arange(BN, dtype=_i32), bonus_batch,
                                      jnp.ones((BN,), _f32), NB, 16 * 1280)
    pool = _outacc(_pack_v(b, with_ones=False), psrc, pdst, pex, BN, NB, 1280, nsplit=1)
    bpool, _ = _unpack_acc(pool, 20)

    vt = _make_scalerows(NB, 20)(bpool, bonus_values_normed.reshape(NB, 1))
    ssrc, sdst, sex, SPE = _pad_edges(bm_col, bm_row, jnp.ones((bm_col.shape[0],), _f32),
                                      n, 16 * 1280)
    sacc = _outacc(_pack_v(vt, with_ones=False), ssrc, sdst, sex, NB, n, 1280, nsplit=25)
    bres, _ = _unpack_acc(sacc, 20)
    return bres


def _pad32(x):
    return jnp.pad(x, ((0, 0), (0, 32 - x.shape[1])))


def kernel(graph_features, graph_edges, bonus_nodes, bonus_edges, bonus_batch, bonus_mapping,
           bonus_values_normed, batch, income, total_armies, params):
    N = graph_features.shape[0]
    BN = bonus_nodes.shape[0]
    NB = bonus_values_normed.shape[0]
    Gn = income.shape[0]

    gsrc = graph_edges[0].astype(_i32)
    gdst = graph_edges[1].astype(_i32)
    eb_true = bonus_edges.shape[1]
    ebp = -(-eb_true // 20480) * 20480 - eb_true
    bsrc = jnp.concatenate([bonus_edges[0].astype(_i32), jnp.zeros((ebp,), _i32)])
    bdst = jnp.concatenate([bonus_edges[1].astype(_i32), jnp.zeros((ebp,), _i32)])
    bm_row = bonus_mapping[0].astype(_i32)
    bm_col = bonus_mapping[1].astype(_i32)
    bnp = jnp.concatenate([bonus_nodes.astype(_i32),
                           jnp.zeros((20480 - BN,), _i32)])

    x = _mm(graph_features, params['init_W'], params['init_b'], act="relu")

    for li, (pb, pg) in enumerate([(params['b1'], params['g1']),
                                   (params['b2'], params['g2']),
                                   (params['b3'], params['g3'])]):
        bres = _bonus_block_sc(_pad32(x), N, bnp, bsrc, bdst, eb_true, bonus_batch,
                               bm_row, bm_col, bonus_values_normed, pb, BN, NB)
        xc = jnp.concatenate([x, bres], axis=1)
        x = _tconv_sc(xc, gsrc, gdst, pg, 800, 400)

    psrc, pdst, pex, PE = _pad_edges(jnp.arange(N, dtype=_i32), batch,
                                     jnp.ones((N,), _f32), Gn, 16 * 1600)
    pool = _outacc(_pack_v(x, with_ones=True), psrc, pdst, pex, N, Gn, 1600, nsplit=1)
    out = _make_head(pool.shape[0], Gn, x.shape[1], 65, 60)(
        pool, income, total_armies, params['f1_W'], params['f1_b'].reshape(1, -1),
        params['f2_W'], params['f2_b'].reshape(1, 1)).reshape(-1)
    pi = jax.nn.log_softmax(jnp.zeros((Gn, 50), dtype=_f32), axis=1)
    return (out, pi)
